# baseline (speedup 1.0000x reference)
import functools
import jax
import jax.numpy as jnp
from jax import lax
from jax.experimental import pallas as pl
from jax.experimental.pallas import tpu as pltpu
from jax.experimental.pallas import tpu_sc as plsc

N = 10000
E = 320000
D = 128
H = 4
C = D // H

ACCW = 144
N_PAD = 10240
NEG = -1e30

NC = 2
NS = 16
NW = NC * NS
EDGES_PER_TILE = E // NW
CHUNK = 80
NCHUNK = EDGES_PER_TILE // CHUNK
ROWS_PER_TILE = N_PAD // NS
ZROWS = 32


def _ln(x, g, b):
    m = jnp.mean(x, axis=-1, keepdims=True)
    v = jnp.var(x, axis=-1, keepdims=True)
    return (x - m) * lax.rsqrt(v + 1e-5) * g + b


def _stage1_body(x_ref, wg_ref, asrc_ref, adst_ref, h_ref, as_ref, ad_ref,
                 ws_ref):
    x = x_ref[...]
    h = jnp.dot(x, wg_ref[...], preferred_element_type=jnp.float32)
    h_ref[...] = h
    rows = lax.broadcasted_iota(jnp.int32, (D, 16), 0) // C
    cols = lax.broadcasted_iota(jnp.int32, (D, 16), 1)
    msel = (rows == cols).astype(jnp.float32)
    a_s = jnp.dot(h * asrc_ref[...], msel, preferred_element_type=jnp.float32)
    a_d = jnp.dot(h * adst_ref[...], msel, preferred_element_type=jnp.float32)
    head = lax.broadcasted_iota(jnp.int32, a_s.shape, 1) < H
    a_s = jnp.where(head, a_s, NEG)
    a_d = jnp.where(head, a_d, NEG)
    as_ref[...] = a_s
    ad_ref[...] = a_d
    t = a_s + a_d
    ws_ref[...] = jnp.exp(jnp.maximum(t, 0.2 * t))


def _stage1(x, W_gat, att_src, att_dst):
    B = 2000
    return pl.pallas_call(
        _stage1_body,
        grid=(N // B,),
        in_specs=[
            pl.BlockSpec((B, D), lambda i: (i, 0)),
            pl.BlockSpec((D, D), lambda i: (0, 0)),
            pl.BlockSpec((D,), lambda i: (0,)),
            pl.BlockSpec((D,), lambda i: (0,)),
        ],
        out_specs=[
            pl.BlockSpec((B, D), lambda i: (i, 0)),
            pl.BlockSpec((B, 16), lambda i: (i, 0)),
            pl.BlockSpec((B, 16), lambda i: (i, 0)),
            pl.BlockSpec((B, 16), lambda i: (i, 0)),
        ],
        out_shape=[
            jax.ShapeDtypeStruct((N, D), jnp.float32),
            jax.ShapeDtypeStruct((N, 16), jnp.float32),
            jax.ShapeDtypeStruct((N, 16), jnp.float32),
            jax.ShapeDtypeStruct((N, 16), jnp.float32),
        ],
    )(x, W_gat, att_src.reshape(D), att_dst.reshape(D))


SUP = 5
NSUP = NCHUNK // SUP


def _bcast_lane(vec, lane):
    idx = jnp.full((16, 1), lane, jnp.int32)
    return lax.gather(
        vec, idx,
        lax.GatherDimensionNumbers(offset_dims=(), collapsed_slice_dims=(0,),
                                   start_index_map=(0,)),
        (1,), mode=lax.GatherScatterMode.PROMISE_IN_BOUNDS)


def _make_stage2():
    mesh = plsc.VectorSubcoreMesh(core_axis_name="c", subcore_axis_name="s")

    bufset = [
        pltpu.VMEM((CHUNK, 16), jnp.float32),
        pltpu.VMEM((CHUNK, 16), jnp.float32),
        pltpu.VMEM((CHUNK, 16), jnp.float32),
        pltpu.VMEM((CHUNK, D), jnp.float32),
        pltpu.SemaphoreType.DMA,
        pltpu.SemaphoreType.DMA,
        pltpu.SemaphoreType.DMA,
        pltpu.SemaphoreType.DMA,
    ]
    idxset = [
        pltpu.VMEM((SUP, CHUNK), jnp.int32),
        pltpu.VMEM((SUP, CHUNK), jnp.int32),
        pltpu.SemaphoreType.DMA,
        pltpu.SemaphoreType.DMA,
    ]

    @functools.partial(
        pl.kernel,
        out_type=[
            jax.ShapeDtypeStruct((NC, N_PAD, D), jnp.float32),
            jax.ShapeDtypeStruct((NC, N_PAD, 16), jnp.float32),
        ],
        mesh=mesh,
        compiler_params=pltpu.CompilerParams(use_tc_tiling_on_sc=False,
                                             needs_layout_passes=False),
        scratch_types=bufset + bufset + idxset + idxset + [
            pltpu.VMEM((ZROWS, D), jnp.float32),
            pltpu.VMEM((ZROWS, 16), jnp.float32),
            pltpu.VMEM_SHARED((N_PAD, D), jnp.float32),
            pltpu.VMEM_SHARED((N_PAD, 16), jnp.float32),
        ],
    )
    def stage2(h_hbm, aspad_hbm, adpad_hbm, src2d_hbm, dst2d_hbm,
               omsg_hbm, oden_hbm,
               asrcA, adstA, wbufA, rinA, semA1, semA2, semA3, semA4,
               asrcB, adstB, wbufB, rinB, semB1, semB2, semB3, semB4,
               sidx0, didx0, is0s, is0d,
               sidx1, didx1, is1s, is1d,
               zbuf_v, zbufd_v, msg_sh, den_sh):
        c = lax.axis_index("c")
        s = lax.axis_index("s")
        wid = c * NS + s
        bufs_a = (asrcA, adstA, wbufA, rinA, semA1, semA2, semA3, semA4)
        bufs_b = (asrcB, adstB, wbufB, rinB, semB1, semB2, semB3, semB4)
        idx0 = (sidx0, didx0, is0s, is0d)
        idx1 = (sidx1, didx1, is1s, is1d)

        def _zrow(i, carry):
            for g in range(D // 16):
                zbuf_v[i, pl.ds(g * 16, 16)] = jnp.zeros((16,), jnp.float32)
            zbufd_v[i, :] = jnp.zeros((16,), jnp.float32)
            return carry
        lax.fori_loop(0, ZROWS, _zrow, 0)
        for k in range(ROWS_PER_TILE // ZROWS):
            off = s * ROWS_PER_TILE + k * ZROWS
            pltpu.sync_copy(zbuf_v, msg_sh.at[pl.ds(off, ZROWS)])
            pltpu.sync_copy(zbufd_v, den_sh.at[pl.ds(off, ZROWS)])
        plsc.subcore_barrier()

        def load_super(sup, ib):
            sb, db, ss, ds_ = ib
            r0 = wid * NCHUNK + sup * SUP
            pltpu.async_copy(src2d_hbm.at[pl.ds(r0, SUP)], sb, ss)
            pltpu.async_copy(dst2d_hbm.at[pl.ds(r0, SUP)], db, ds_)

        def wait_super(sup, ib):
            sb, db, ss, ds_ = ib
            r0 = wid * NCHUNK + sup * SUP
            pltpu.make_async_copy(src2d_hbm.at[pl.ds(r0, SUP)], sb, ss).wait()
            pltpu.make_async_copy(dst2d_hbm.at[pl.ds(r0, SUP)], db, ds_).wait()

        def issue(srow, drow, bufs):
            asrc_v, adst_v, _, rin_v, s1, s2, s3, s4 = bufs
            pltpu.async_copy(h_hbm.at[srow], rin_v, s1)
            pltpu.async_copy(aspad_hbm.at[srow], asrc_v, s2)
            pltpu.async_copy(adpad_hbm.at[drow], adst_v, s3)

        def process(srow, drow, bufs, dsem_wait):
            asrc_v, adst_v, wbuf_v, rin_v, s1, s2, s3, s4 = bufs
            if dsem_wait is not None:
                pltpu.make_async_copy(wbuf_v, den_sh.at[dsem_wait], s4).wait()
            pltpu.make_async_copy(aspad_hbm.at[srow], asrc_v, s2).wait()
            pltpu.make_async_copy(adpad_hbm.at[drow], adst_v, s3).wait()
            pltpu.make_async_copy(h_hbm.at[srow], rin_v, s1).wait()

            @plsc.parallel_loop(0, CHUNK, unroll=8)
            def _edge(e):
                t = asrc_v[e, :] + adst_v[e, :]
                w = jnp.exp(jnp.maximum(t, 0.2 * t))
                wbuf_v[e, :] = w
                for hh in range(H):
                    w_spl = _bcast_lane(w, hh)
                    for g in range(2):
                        c0 = hh * C + g * 16
                        rin_v[e, pl.ds(c0, 16)] = (
                            rin_v[e, pl.ds(c0, 16)] * w_spl)

            pltpu.async_copy(wbuf_v, den_sh.at[drow], s4, add=True)
            pltpu.sync_copy(rin_v, msg_sh.at[drow], add=True)

        wait0 = load_super(0, idx0)
        wait_super(0, idx0)
        issue(sidx0.at[0], didx0.at[0], bufs_a)

        def body(j, carry, first=False):
            load_super(2 * j + 1, idx1)
            for k in range(10):
                half = k // 5
                r = k % 5
                ib = idx0 if half == 0 else idx1
                cur = bufs_a if k % 2 == 0 else bufs_b
                nxt = bufs_b if k % 2 == 0 else bufs_a
                if k == 4:
                    wait_super(2 * j + 1, idx1)
                if k == 5:
                    load_super(2 * j + 2, idx0)
                if k == 9:
                    wait_super(2 * j + 2, idx0)
                nk = k + 1
                nib = idx0 if (nk // 5) % 2 == 0 else idx1
                nr = nk % 5
                issue(nib[0].at[nr], nib[1].at[nr], nxt)
                pk = k - 2
                if pk >= 0:
                    prow = (idx0 if (pk // 5) == 0 else idx1)[1].at[pk % 5]
                else:
                    prow = (idx1 if (pk + 10) // 5 == 1 else idx0)[1].at[
                        (pk + 10) % 5]
                pdrow = None if first and k < 2 else prow
                process(ib[0].at[r], ib[1].at[r], cur, pdrow)
            return carry

        body(0, 0, first=True)
        lax.fori_loop(1, (NSUP - 1) // 2, body, 0)

        for k in range(SUP):
            cur = bufs_a if k % 2 == 0 else bufs_b
            nxt = bufs_b if k % 2 == 0 else bufs_a
            if k < SUP - 1:
                issue(sidx0.at[k + 1], didx0.at[k + 1], nxt)
            pk = k - 2
            if pk >= 0:
                prow = didx0.at[pk]
            else:
                prow = didx1.at[pk + 10 - 5]
            process(sidx0.at[k], didx0.at[k], cur, prow)
        pltpu.make_async_copy(bufs_b[2], den_sh.at[didx0.at[SUP - 2]],
                              bufs_b[7]).wait()
        pltpu.make_async_copy(bufs_a[2], den_sh.at[didx0.at[SUP - 1]],
                              bufs_a[7]).wait()

        plsc.subcore_barrier()
        off = s * ROWS_PER_TILE
        pltpu.sync_copy(msg_sh.at[pl.ds(off, ROWS_PER_TILE)],
                        omsg_hbm.at[c, pl.ds(off, ROWS_PER_TILE)])
        pltpu.sync_copy(den_sh.at[pl.ds(off, ROWS_PER_TILE)],
                        oden_hbm.at[c, pl.ds(off, ROWS_PER_TILE)])

    return stage2


_stage2 = _make_stage2()


def _stage3_body(x_ref, h_ref, ws_ref, pm0_ref, pm1_ref, pd0_ref, pd1_ref,
                 bias_ref, gamma_ref, beta_ref, W1_ref, b1_ref, W2_ref,
                 b2_ref, out_ref):
    x = x_ref[...]
    h = h_ref[...]
    ws = ws_ref[...]
    rows = lax.broadcasted_iota(jnp.int32, (16, D), 0)
    cols = lax.broadcasted_iota(jnp.int32, (16, D), 1) // C
    m16 = (rows == cols).astype(jnp.float32)
    den16 = pd0_ref[0] + pd1_ref[0] + ws
    den = jnp.dot(den16, m16, preferred_element_type=jnp.float32)
    ws128 = jnp.dot(ws, m16, preferred_element_type=jnp.float32)
    acc = pm0_ref[0] + pm1_ref[0] + h * ws128
    gat = acc / (den + 1e-16) + bias_ref[...]
    gamma = gamma_ref[...]
    beta = beta_ref[...]
    x1 = _ln(x + gat, gamma, beta)
    hmid = jnp.maximum(
        jnp.dot(x1, W1_ref[...], preferred_element_type=jnp.float32)
        + b1_ref[...], 0.0)
    h2 = jnp.dot(hmid, W2_ref[...], preferred_element_type=jnp.float32) \
        + b2_ref[...]
    out_ref[...] = _ln(x1 + h2, gamma, beta)


def _stage3(x, h, ws, pmsg, pden, bias_gat, gamma, beta, W1, b1, W2, b2):
    B = 2000
    return pl.pallas_call(
        _stage3_body,
        grid=(N // B,),
        in_specs=[
            pl.BlockSpec((B, D), lambda i: (i, 0)),
            pl.BlockSpec((B, D), lambda i: (i, 0)),
            pl.BlockSpec((B, 16), lambda i: (i, 0)),
            pl.BlockSpec((1, B, D), lambda i: (0, i, 0)),
            pl.BlockSpec((1, B, D), lambda i: (1, i, 0)),
            pl.BlockSpec((1, B, 16), lambda i: (0, i, 0)),
            pl.BlockSpec((1, B, 16), lambda i: (1, i, 0)),
            pl.BlockSpec((D,), lambda i: (0,)),
            pl.BlockSpec((D,), lambda i: (0,)),
            pl.BlockSpec((D,), lambda i: (0,)),
            pl.BlockSpec((D, 2 * D), lambda i: (0, 0)),
            pl.BlockSpec((2 * D,), lambda i: (0,)),
            pl.BlockSpec((2 * D, D), lambda i: (0, 0)),
            pl.BlockSpec((D,), lambda i: (0,)),
        ],
        out_specs=pl.BlockSpec((B, D), lambda i: (i, 0)),
        out_shape=jax.ShapeDtypeStruct((N, D), jnp.float32),
    )(x, h, ws, pmsg, pmsg, pden, pden, bias_gat, gamma, beta,
      W1, b1, W2, b2)


def kernel(x, edge_index, W_gat, att_src, att_dst, bias_gat, gamma, beta,
           W1, b1, W2, b2):
    h, aspad, adpad, wself = _stage1(x, W_gat, att_src, att_dst)
    src = edge_index[0].reshape(E // CHUNK, CHUNK)
    dst = edge_index[1].reshape(E // CHUNK, CHUNK)
    pmsg, pden = _stage2(h, aspad, adpad, src, dst)
    return _stage3(x, h, wself, pmsg, pden, bias_gat, gamma, beta,
                   W1, b1, W2, b2)


# speedup vs baseline: 1.0634x; 1.0634x over previous
import functools
import jax
import jax.numpy as jnp
from jax import lax
from jax.experimental import pallas as pl
from jax.experimental.pallas import tpu as pltpu
from jax.experimental.pallas import tpu_sc as plsc

N = 10000
E = 320000
D = 128
H = 4
C = D // H

ACCW = 144
N_PAD = 10240
NEG = -1e30

NC = 2
NS = 16
NW = NC * NS
EDGES_PER_TILE = E // NW
CHUNK = 80
NCHUNK = EDGES_PER_TILE // CHUNK
ROWS_PER_TILE = N_PAD // NS
ZROWS = 32


def _ln(x, g, b):
    m = jnp.mean(x, axis=-1, keepdims=True)
    v = jnp.var(x, axis=-1, keepdims=True)
    return (x - m) * lax.rsqrt(v + 1e-5) * g + b


def _stage1_body(x_ref, wg_ref, asrc_ref, adst_ref, h_ref, hbf_ref, as_ref,
                 ad_ref, ws_ref):
    x = x_ref[...]
    h = jnp.dot(x, wg_ref[...], preferred_element_type=jnp.float32)
    h_ref[...] = h
    jj = lax.broadcasted_iota(jnp.int32, (D, D), 0)
    ss = lax.broadcasted_iota(jnp.int32, (D, D), 1)
    r = ss % C
    srcc = (ss // C) * C + jnp.where(r % 2 == 0, r // 2, C // 2 + r // 2)
    pmat = (jj == srcc).astype(jnp.float32)
    hbf_ref[...] = jnp.dot(h, pmat,
                           preferred_element_type=jnp.float32).astype(
                               jnp.bfloat16)
    rows = lax.broadcasted_iota(jnp.int32, (D, 16), 0) // C
    cols = lax.broadcasted_iota(jnp.int32, (D, 16), 1)
    msel = (rows == cols).astype(jnp.float32)
    a_s = jnp.dot(h * asrc_ref[...], msel, preferred_element_type=jnp.float32)
    a_d = jnp.dot(h * adst_ref[...], msel, preferred_element_type=jnp.float32)
    head = lax.broadcasted_iota(jnp.int32, a_s.shape, 1) < H
    a_s = jnp.where(head, a_s, NEG)
    a_d = jnp.where(head, a_d, NEG)
    as_ref[...] = a_s
    ad_ref[...] = a_d
    t = a_s + a_d
    ws_ref[...] = jnp.exp(jnp.maximum(t, 0.2 * t))


def _stage1(x, W_gat, att_src, att_dst):
    B = 2000
    return pl.pallas_call(
        _stage1_body,
        grid=(N // B,),
        in_specs=[
            pl.BlockSpec((B, D), lambda i: (i, 0)),
            pl.BlockSpec((D, D), lambda i: (0, 0)),
            pl.BlockSpec((D,), lambda i: (0,)),
            pl.BlockSpec((D,), lambda i: (0,)),
        ],
        out_specs=[
            pl.BlockSpec((B, D), lambda i: (i, 0)),
            pl.BlockSpec((B, D), lambda i: (i, 0)),
            pl.BlockSpec((B, 16), lambda i: (i, 0)),
            pl.BlockSpec((B, 16), lambda i: (i, 0)),
            pl.BlockSpec((B, 16), lambda i: (i, 0)),
        ],
        out_shape=[
            jax.ShapeDtypeStruct((N, D), jnp.float32),
            jax.ShapeDtypeStruct((N, D), jnp.bfloat16),
            jax.ShapeDtypeStruct((N, 16), jnp.float32),
            jax.ShapeDtypeStruct((N, 16), jnp.float32),
            jax.ShapeDtypeStruct((N, 16), jnp.float32),
        ],
    )(x, W_gat, att_src.reshape(D), att_dst.reshape(D))


SUP = 5
NSUP = NCHUNK // SUP


def _bcast_lane(vec, lane):
    idx = jnp.full((16, 1), lane, jnp.int32)
    return lax.gather(
        vec, idx,
        lax.GatherDimensionNumbers(offset_dims=(), collapsed_slice_dims=(0,),
                                   start_index_map=(0,)),
        (1,), mode=lax.GatherScatterMode.PROMISE_IN_BOUNDS)


def _make_stage2():
    mesh = plsc.VectorSubcoreMesh(core_axis_name="c", subcore_axis_name="s")

    bufset = [
        pltpu.VMEM((CHUNK, 16), jnp.float32),
        pltpu.VMEM((CHUNK, 16), jnp.float32),
        pltpu.VMEM((CHUNK, 16), jnp.float32),
        pltpu.VMEM((CHUNK, D), jnp.bfloat16),
        pltpu.SemaphoreType.DMA,
        pltpu.SemaphoreType.DMA,
        pltpu.SemaphoreType.DMA,
        pltpu.SemaphoreType.DMA,
    ]
    idxset = [
        pltpu.VMEM((SUP, CHUNK), jnp.int32),
        pltpu.VMEM((SUP, CHUNK), jnp.int32),
        pltpu.SemaphoreType.DMA,
        pltpu.SemaphoreType.DMA,
    ]

    @functools.partial(
        pl.kernel,
        out_type=[
            jax.ShapeDtypeStruct((NC, N_PAD, D), jnp.float32),
            jax.ShapeDtypeStruct((NC, N_PAD, 16), jnp.float32),
        ],
        mesh=mesh,
        compiler_params=pltpu.CompilerParams(use_tc_tiling_on_sc=False,
                                             needs_layout_passes=False),
        scratch_types=bufset + bufset + idxset + idxset + [
            pltpu.VMEM((CHUNK, D), jnp.float32),
            pltpu.VMEM((ZROWS, D), jnp.float32),
            pltpu.VMEM((ZROWS, 16), jnp.float32),
            pltpu.VMEM_SHARED((N_PAD, D), jnp.float32),
            pltpu.VMEM_SHARED((N_PAD, 16), jnp.float32),
        ],
    )
    def stage2(h_hbm, aspad_hbm, adpad_hbm, src2d_hbm, dst2d_hbm,
               omsg_hbm, oden_hbm,
               asrcA, adstA, wbufA, rinA, semA1, semA2, semA3, semA4,
               asrcB, adstB, wbufB, rinB, semB1, semB2, semB3, semB4,
               sidx0, didx0, is0s, is0d,
               sidx1, didx1, is1s, is1d,
               rout_v, zbuf_v, zbufd_v, msg_sh, den_sh):
        c = lax.axis_index("c")
        s = lax.axis_index("s")
        wid = c * NS + s
        bufs_a = (asrcA, adstA, wbufA, rinA, semA1, semA2, semA3, semA4)
        bufs_b = (asrcB, adstB, wbufB, rinB, semB1, semB2, semB3, semB4)
        idx0 = (sidx0, didx0, is0s, is0d)
        idx1 = (sidx1, didx1, is1s, is1d)

        def _zrow(i, carry):
            for g in range(D // 16):
                zbuf_v[i, pl.ds(g * 16, 16)] = jnp.zeros((16,), jnp.float32)
            zbufd_v[i, :] = jnp.zeros((16,), jnp.float32)
            return carry
        lax.fori_loop(0, ZROWS, _zrow, 0)
        for k in range(ROWS_PER_TILE // ZROWS):
            off = s * ROWS_PER_TILE + k * ZROWS
            pltpu.sync_copy(zbuf_v, msg_sh.at[pl.ds(off, ZROWS)])
            pltpu.sync_copy(zbufd_v, den_sh.at[pl.ds(off, ZROWS)])
        plsc.subcore_barrier()

        def load_super(sup, ib):
            sb, db, ss, ds_ = ib
            r0 = wid * NCHUNK + sup * SUP
            pltpu.async_copy(src2d_hbm.at[pl.ds(r0, SUP)], sb, ss)
            pltpu.async_copy(dst2d_hbm.at[pl.ds(r0, SUP)], db, ds_)

        def wait_super(sup, ib):
            sb, db, ss, ds_ = ib
            r0 = wid * NCHUNK + sup * SUP
            pltpu.make_async_copy(src2d_hbm.at[pl.ds(r0, SUP)], sb, ss).wait()
            pltpu.make_async_copy(dst2d_hbm.at[pl.ds(r0, SUP)], db, ds_).wait()

        def issue(srow, drow, bufs):
            asrc_v, adst_v, _, rin_v, s1, s2, s3, s4 = bufs
            pltpu.async_copy(h_hbm.at[srow], rin_v, s1)
            pltpu.async_copy(aspad_hbm.at[srow], asrc_v, s2)
            pltpu.async_copy(adpad_hbm.at[drow], adst_v, s3)

        def process(srow, drow, bufs, dsem_wait):
            asrc_v, adst_v, wbuf_v, rin_v, s1, s2, s3, s4 = bufs
            if dsem_wait is not None:
                pltpu.make_async_copy(wbuf_v, den_sh.at[dsem_wait], s4).wait()
            pltpu.make_async_copy(aspad_hbm.at[srow], asrc_v, s2).wait()
            pltpu.make_async_copy(adpad_hbm.at[drow], adst_v, s3).wait()
            pltpu.make_async_copy(h_hbm.at[srow], rin_v, s1).wait()

            @plsc.parallel_loop(0, CHUNK, unroll=4)
            def _edge(e):
                t = asrc_v[e, :] + adst_v[e, :]
                w = jnp.exp(jnp.maximum(t, 0.2 * t))
                wbuf_v[e, :] = w
                for hh in range(H):
                    w_spl = _bcast_lane(w, hh)
                    xa, xb = plsc.unpack(
                        rin_v[e, pl.ds(hh * C, C)],
                        format=plsc.PackFormat.INTERLEAVED)
                    rout_v[e, pl.ds(hh * C, 16)] = xa * w_spl
                    rout_v[e, pl.ds(hh * C + 16, 16)] = xb * w_spl

            pltpu.async_copy(wbuf_v, den_sh.at[drow], s4, add=True)
            pltpu.sync_copy(rout_v, msg_sh.at[drow], add=True)

        wait0 = load_super(0, idx0)
        wait_super(0, idx0)
        issue(sidx0.at[0], didx0.at[0], bufs_a)

        def body(j, carry, first=False):
            load_super(2 * j + 1, idx1)
            for k in range(10):
                half = k // 5
                r = k % 5
                ib = idx0 if half == 0 else idx1
                cur = bufs_a if k % 2 == 0 else bufs_b
                nxt = bufs_b if k % 2 == 0 else bufs_a
                if k == 4:
                    wait_super(2 * j + 1, idx1)
                if k == 5:
                    load_super(2 * j + 2, idx0)
                if k == 9:
                    wait_super(2 * j + 2, idx0)
                nk = k + 1
                nib = idx0 if (nk // 5) % 2 == 0 else idx1
                nr = nk % 5
                issue(nib[0].at[nr], nib[1].at[nr], nxt)
                pk = k - 2
                if pk >= 0:
                    prow = (idx0 if (pk // 5) == 0 else idx1)[1].at[pk % 5]
                else:
                    prow = (idx1 if (pk + 10) // 5 == 1 else idx0)[1].at[
                        (pk + 10) % 5]
                pdrow = None if first and k < 2 else prow
                process(ib[0].at[r], ib[1].at[r], cur, pdrow)
            return carry

        body(0, 0, first=True)
        lax.fori_loop(1, (NSUP - 1) // 2, body, 0)

        for k in range(SUP):
            cur = bufs_a if k % 2 == 0 else bufs_b
            nxt = bufs_b if k % 2 == 0 else bufs_a
            if k < SUP - 1:
                issue(sidx0.at[k + 1], didx0.at[k + 1], nxt)
            pk = k - 2
            if pk >= 0:
                prow = didx0.at[pk]
            else:
                prow = didx1.at[pk + 10 - 5]
            process(sidx0.at[k], didx0.at[k], cur, prow)
        pltpu.make_async_copy(bufs_b[2], den_sh.at[didx0.at[SUP - 2]],
                              bufs_b[7]).wait()
        pltpu.make_async_copy(bufs_a[2], den_sh.at[didx0.at[SUP - 1]],
                              bufs_a[7]).wait()

        plsc.subcore_barrier()
        off = s * ROWS_PER_TILE
        pltpu.sync_copy(msg_sh.at[pl.ds(off, ROWS_PER_TILE)],
                        omsg_hbm.at[c, pl.ds(off, ROWS_PER_TILE)])
        pltpu.sync_copy(den_sh.at[pl.ds(off, ROWS_PER_TILE)],
                        oden_hbm.at[c, pl.ds(off, ROWS_PER_TILE)])

    return stage2


_stage2 = _make_stage2()


def _stage3_body(x_ref, h_ref, ws_ref, pm0_ref, pm1_ref, pd0_ref, pd1_ref,
                 bias_ref, gamma_ref, beta_ref, W1_ref, b1_ref, W2_ref,
                 b2_ref, out_ref):
    x = x_ref[...]
    h = h_ref[...]
    ws = ws_ref[...]
    rows = lax.broadcasted_iota(jnp.int32, (16, D), 0)
    cols = lax.broadcasted_iota(jnp.int32, (16, D), 1) // C
    m16 = (rows == cols).astype(jnp.float32)
    den16 = pd0_ref[0] + pd1_ref[0] + ws
    den = jnp.dot(den16, m16, preferred_element_type=jnp.float32)
    ws128 = jnp.dot(ws, m16, preferred_element_type=jnp.float32)
    acc = pm0_ref[0] + pm1_ref[0] + h * ws128
    gat = acc / (den + 1e-16) + bias_ref[...]
    gamma = gamma_ref[...]
    beta = beta_ref[...]
    x1 = _ln(x + gat, gamma, beta)
    hmid = jnp.maximum(
        jnp.dot(x1, W1_ref[...], preferred_element_type=jnp.float32)
        + b1_ref[...], 0.0)
    h2 = jnp.dot(hmid, W2_ref[...], preferred_element_type=jnp.float32) \
        + b2_ref[...]
    out_ref[...] = _ln(x1 + h2, gamma, beta)


def _stage3(x, h, ws, pmsg, pden, bias_gat, gamma, beta, W1, b1, W2, b2):
    B = 2000
    return pl.pallas_call(
        _stage3_body,
        grid=(N // B,),
        in_specs=[
            pl.BlockSpec((B, D), lambda i: (i, 0)),
            pl.BlockSpec((B, D), lambda i: (i, 0)),
            pl.BlockSpec((B, 16), lambda i: (i, 0)),
            pl.BlockSpec((1, B, D), lambda i: (0, i, 0)),
            pl.BlockSpec((1, B, D), lambda i: (1, i, 0)),
            pl.BlockSpec((1, B, 16), lambda i: (0, i, 0)),
            pl.BlockSpec((1, B, 16), lambda i: (1, i, 0)),
            pl.BlockSpec((D,), lambda i: (0,)),
            pl.BlockSpec((D,), lambda i: (0,)),
            pl.BlockSpec((D,), lambda i: (0,)),
            pl.BlockSpec((D, 2 * D), lambda i: (0, 0)),
            pl.BlockSpec((2 * D,), lambda i: (0,)),
            pl.BlockSpec((2 * D, D), lambda i: (0, 0)),
            pl.BlockSpec((D,), lambda i: (0,)),
        ],
        out_specs=pl.BlockSpec((B, D), lambda i: (i, 0)),
        out_shape=jax.ShapeDtypeStruct((N, D), jnp.float32),
    )(x, h, ws, pmsg, pmsg, pden, pden, bias_gat, gamma, beta,
      W1, b1, W2, b2)


def kernel(x, edge_index, W_gat, att_src, att_dst, bias_gat, gamma, beta,
           W1, b1, W2, b2):
    h, h_bf, aspad, adpad, wself = _stage1(x, W_gat, att_src, att_dst)
    src = edge_index[0].reshape(E // CHUNK, CHUNK)
    dst = edge_index[1].reshape(E // CHUNK, CHUNK)
    pmsg, pden = _stage2(h_bf, aspad, adpad, src, dst)
    return _stage3(x, h, wself, pmsg, pden, bias_gat, gamma, beta,
                   W1, b1, W2, b2)


# speedup vs baseline: 1.1210x; 1.0542x over previous
import functools
import jax
import jax.numpy as jnp
from jax import lax
from jax.experimental import pallas as pl
from jax.experimental.pallas import tpu as pltpu
from jax.experimental.pallas import tpu_sc as plsc

N = 10000
E = 320000
D = 128
H = 4
C = D // H

ACCW = 144
N_PAD = 10112
NEG = -1e30

NC = 2
NS = 16
NW = NC * NS
EDGES_PER_TILE = E // NW
CHUNK = 80
NCHUNK = EDGES_PER_TILE // CHUNK
ROWS_PER_TILE = N_PAD // NS
ZROWS = 32


def _ln(x, g, b):
    m = jnp.mean(x, axis=-1, keepdims=True)
    v = jnp.var(x, axis=-1, keepdims=True)
    return (x - m) * lax.rsqrt(v + 1e-5) * g + b


def _stage1_body(x_ref, wg_ref, asrc_ref, adst_ref, h_ref, hbf_ref, as_ref,
                 ad_ref, ws_ref):
    x = x_ref[...]
    h = jnp.dot(x, wg_ref[...], preferred_element_type=jnp.float32)
    h_ref[...] = h
    jj = lax.broadcasted_iota(jnp.int32, (D, D), 0)
    ss = lax.broadcasted_iota(jnp.int32, (D, D), 1)
    r = ss % C
    srcc = (ss // C) * C + jnp.where(r % 2 == 0, r // 2, C // 2 + r // 2)
    pmat = (jj == srcc).astype(jnp.float32)
    hbf_ref[...] = jnp.dot(h, pmat,
                           preferred_element_type=jnp.float32).astype(
                               jnp.bfloat16)
    rows = lax.broadcasted_iota(jnp.int32, (D, 16), 0) // C
    cols = lax.broadcasted_iota(jnp.int32, (D, 16), 1)
    msel = (rows == cols).astype(jnp.float32)
    a_s = jnp.dot(h * asrc_ref[...], msel, preferred_element_type=jnp.float32)
    a_d = jnp.dot(h * adst_ref[...], msel, preferred_element_type=jnp.float32)
    head = lax.broadcasted_iota(jnp.int32, a_s.shape, 1) < H
    a_s = jnp.where(head, a_s, NEG)
    a_d = jnp.where(head, a_d, NEG)
    as_ref[...] = a_s
    ad_ref[...] = a_d
    t = a_s + a_d
    ws_ref[...] = jnp.exp(jnp.maximum(t, 0.2 * t))


def _stage1(x, W_gat, att_src, att_dst):
    B = 2000
    return pl.pallas_call(
        _stage1_body,
        grid=(N // B,),
        in_specs=[
            pl.BlockSpec((B, D), lambda i: (i, 0)),
            pl.BlockSpec((D, D), lambda i: (0, 0)),
            pl.BlockSpec((D,), lambda i: (0,)),
            pl.BlockSpec((D,), lambda i: (0,)),
        ],
        out_specs=[
            pl.BlockSpec((B, D), lambda i: (i, 0)),
            pl.BlockSpec((B, D), lambda i: (i, 0)),
            pl.BlockSpec((B, 16), lambda i: (i, 0)),
            pl.BlockSpec((B, 16), lambda i: (i, 0)),
            pl.BlockSpec((B, 16), lambda i: (i, 0)),
        ],
        out_shape=[
            jax.ShapeDtypeStruct((N, D), jnp.float32),
            jax.ShapeDtypeStruct((N, D), jnp.bfloat16),
            jax.ShapeDtypeStruct((N, 16), jnp.float32),
            jax.ShapeDtypeStruct((N, 16), jnp.float32),
            jax.ShapeDtypeStruct((N, 16), jnp.float32),
        ],
    )(x, W_gat, att_src.reshape(D), att_dst.reshape(D))


SUP = 5
NSUP = NCHUNK // SUP


def _bcast_lane(vec, lane):
    idx = jnp.full((16, 1), lane, jnp.int32)
    return lax.gather(
        vec, idx,
        lax.GatherDimensionNumbers(offset_dims=(), collapsed_slice_dims=(0,),
                                   start_index_map=(0,)),
        (1,), mode=lax.GatherScatterMode.PROMISE_IN_BOUNDS)


def _make_stage2():
    mesh = plsc.VectorSubcoreMesh(core_axis_name="c", subcore_axis_name="s")

    bufset = [
        pltpu.VMEM((CHUNK, 16), jnp.float32),
        pltpu.VMEM((CHUNK, 16), jnp.float32),
        pltpu.VMEM((CHUNK, 16), jnp.float32),
        pltpu.VMEM((CHUNK, D), jnp.bfloat16),
        pltpu.SemaphoreType.DMA,
        pltpu.SemaphoreType.DMA,
        pltpu.SemaphoreType.DMA,
        pltpu.SemaphoreType.DMA,
    ]
    idxset = [
        pltpu.VMEM((SUP, CHUNK), jnp.int32),
        pltpu.VMEM((SUP, CHUNK), jnp.int32),
        pltpu.SemaphoreType.DMA,
        pltpu.SemaphoreType.DMA,
    ]

    @functools.partial(
        pl.kernel,
        out_type=[
            jax.ShapeDtypeStruct((NC, N_PAD, D), jnp.float32),
            jax.ShapeDtypeStruct((NC, N_PAD, 16), jnp.float32),
        ],
        mesh=mesh,
        compiler_params=pltpu.CompilerParams(use_tc_tiling_on_sc=False,
                                             needs_layout_passes=False),
        scratch_types=bufset + bufset + idxset + idxset + [
            pltpu.VMEM((CHUNK, D), jnp.float32),
            pltpu.VMEM((CHUNK, D), jnp.float32),
            pltpu.SemaphoreType.DMA,
            pltpu.SemaphoreType.DMA,
            pltpu.VMEM_SHARED((N_PAD, D), jnp.float32),
            pltpu.VMEM_SHARED((N_PAD, 16), jnp.float32),
        ],
    )
    def stage2(h_hbm, aspad_hbm, adpad_hbm, src2d_hbm, dst2d_hbm,
               zmsg_hbm, zden_hbm,
               omsg_hbm, oden_hbm,
               asrcA, adstA, wbufA, rinA, semA1, semA2, semA3, semA4,
               asrcB, adstB, wbufB, rinB, semB1, semB2, semB3, semB4,
               sidx0, didx0, is0s, is0d,
               sidx1, didx1, is1s, is1d,
               routA, routB, smA, smB, msg_sh, den_sh):
        c = lax.axis_index("c")
        s = lax.axis_index("s")
        wid = c * NS + s
        bufs_a = (asrcA, adstA, wbufA, rinA, semA1, semA2, semA3, semA4,
                  routA, smA)
        bufs_b = (asrcB, adstB, wbufB, rinB, semB1, semB2, semB3, semB4,
                  routB, smB)
        idx0 = (sidx0, didx0, is0s, is0d)
        idx1 = (sidx1, didx1, is1s, is1d)

        off = s * ROWS_PER_TILE
        pltpu.sync_copy(zmsg_hbm.at[pl.ds(off, ROWS_PER_TILE)],
                        msg_sh.at[pl.ds(off, ROWS_PER_TILE)])
        pltpu.sync_copy(zden_hbm.at[pl.ds(off, ROWS_PER_TILE)],
                        den_sh.at[pl.ds(off, ROWS_PER_TILE)])
        plsc.subcore_barrier()

        def load_super(sup, ib):
            sb, db, ss, ds_ = ib
            r0 = wid * NCHUNK + sup * SUP
            pltpu.async_copy(src2d_hbm.at[pl.ds(r0, SUP)], sb, ss)
            pltpu.async_copy(dst2d_hbm.at[pl.ds(r0, SUP)], db, ds_)

        def wait_super(sup, ib):
            sb, db, ss, ds_ = ib
            r0 = wid * NCHUNK + sup * SUP
            pltpu.make_async_copy(src2d_hbm.at[pl.ds(r0, SUP)], sb, ss).wait()
            pltpu.make_async_copy(dst2d_hbm.at[pl.ds(r0, SUP)], db, ds_).wait()

        def issue(srow, drow, bufs):
            asrc_v, adst_v, _, rin_v, s1, s2, s3, s4 = bufs[:8]
            pltpu.async_copy(h_hbm.at[srow], rin_v, s1)
            pltpu.async_copy(aspad_hbm.at[srow], asrc_v, s2)
            pltpu.async_copy(adpad_hbm.at[drow], adst_v, s3)

        def process(srow, drow, bufs, dsem_wait):
            asrc_v, adst_v, wbuf_v, rin_v, s1, s2, s3, s4, rout_v, sm = bufs
            if dsem_wait is not None:
                pltpu.make_async_copy(wbuf_v, den_sh.at[dsem_wait], s4).wait()
                pltpu.make_async_copy(rout_v, msg_sh.at[dsem_wait], sm).wait()
            pltpu.make_async_copy(aspad_hbm.at[srow], asrc_v, s2).wait()
            pltpu.make_async_copy(adpad_hbm.at[drow], adst_v, s3).wait()
            pltpu.make_async_copy(h_hbm.at[srow], rin_v, s1).wait()

            @plsc.parallel_loop(0, CHUNK, unroll=4)
            def _edge(e):
                t = asrc_v[e, :] + adst_v[e, :]
                w = jnp.exp(jnp.maximum(t, 0.2 * t))
                wbuf_v[e, :] = w
                for hh in range(H):
                    w_spl = _bcast_lane(w, hh)
                    xa, xb = plsc.unpack(
                        rin_v[e, pl.ds(hh * C, C)],
                        format=plsc.PackFormat.INTERLEAVED)
                    rout_v[e, pl.ds(hh * C, 16)] = xa * w_spl
                    rout_v[e, pl.ds(hh * C + 16, 16)] = xb * w_spl

            pltpu.async_copy(wbuf_v, den_sh.at[drow], s4, add=True)
            pltpu.async_copy(rout_v, msg_sh.at[drow], sm, add=True)

        wait0 = load_super(0, idx0)
        wait_super(0, idx0)
        issue(sidx0.at[0], didx0.at[0], bufs_a)

        def body(j, carry, first=False):
            load_super(2 * j + 1, idx1)
            for k in range(10):
                half = k // 5
                r = k % 5
                ib = idx0 if half == 0 else idx1
                cur = bufs_a if k % 2 == 0 else bufs_b
                nxt = bufs_b if k % 2 == 0 else bufs_a
                if k == 4:
                    wait_super(2 * j + 1, idx1)
                if k == 5:
                    load_super(2 * j + 2, idx0)
                if k == 9:
                    wait_super(2 * j + 2, idx0)
                nk = k + 1
                nib = idx0 if (nk // 5) % 2 == 0 else idx1
                nr = nk % 5
                issue(nib[0].at[nr], nib[1].at[nr], nxt)
                pk = k - 2
                if pk >= 0:
                    prow = (idx0 if (pk // 5) == 0 else idx1)[1].at[pk % 5]
                else:
                    prow = (idx1 if (pk + 10) // 5 == 1 else idx0)[1].at[
                        (pk + 10) % 5]
                pdrow = None if first and k < 2 else prow
                process(ib[0].at[r], ib[1].at[r], cur, pdrow)
            return carry

        body(0, 0, first=True)
        lax.fori_loop(1, (NSUP - 1) // 2, body, 0)

        for k in range(SUP):
            cur = bufs_a if k % 2 == 0 else bufs_b
            nxt = bufs_b if k % 2 == 0 else bufs_a
            if k < SUP - 1:
                issue(sidx0.at[k + 1], didx0.at[k + 1], nxt)
            pk = k - 2
            if pk >= 0:
                prow = didx0.at[pk]
            else:
                prow = didx1.at[pk + 10 - 5]
            process(sidx0.at[k], didx0.at[k], cur, prow)
        pltpu.make_async_copy(bufs_b[2], den_sh.at[didx0.at[SUP - 2]],
                              bufs_b[7]).wait()
        pltpu.make_async_copy(bufs_a[2], den_sh.at[didx0.at[SUP - 1]],
                              bufs_a[7]).wait()
        pltpu.make_async_copy(bufs_b[8], msg_sh.at[didx0.at[SUP - 2]],
                              bufs_b[9]).wait()
        pltpu.make_async_copy(bufs_a[8], msg_sh.at[didx0.at[SUP - 1]],
                              bufs_a[9]).wait()

        plsc.subcore_barrier()
        off = s * ROWS_PER_TILE
        pltpu.sync_copy(msg_sh.at[pl.ds(off, ROWS_PER_TILE)],
                        omsg_hbm.at[c, pl.ds(off, ROWS_PER_TILE)])
        pltpu.sync_copy(den_sh.at[pl.ds(off, ROWS_PER_TILE)],
                        oden_hbm.at[c, pl.ds(off, ROWS_PER_TILE)])

    return stage2


_stage2 = _make_stage2()


def _stage3_body(x_ref, h_ref, ws_ref, pm0_ref, pm1_ref, pd0_ref, pd1_ref,
                 bias_ref, gamma_ref, beta_ref, W1_ref, b1_ref, W2_ref,
                 b2_ref, out_ref):
    x = x_ref[...]
    h = h_ref[...]
    ws = ws_ref[...]
    rows = lax.broadcasted_iota(jnp.int32, (16, D), 0)
    cols = lax.broadcasted_iota(jnp.int32, (16, D), 1) // C
    m16 = (rows == cols).astype(jnp.float32)
    den16 = pd0_ref[0] + pd1_ref[0] + ws
    den = jnp.dot(den16, m16, preferred_element_type=jnp.float32)
    ws128 = jnp.dot(ws, m16, preferred_element_type=jnp.float32)
    acc = pm0_ref[0] + pm1_ref[0] + h * ws128
    gat = acc / (den + 1e-16) + bias_ref[...]
    gamma = gamma_ref[...]
    beta = beta_ref[...]
    x1 = _ln(x + gat, gamma, beta)
    hmid = jnp.maximum(
        jnp.dot(x1, W1_ref[...], preferred_element_type=jnp.float32)
        + b1_ref[...], 0.0)
    h2 = jnp.dot(hmid, W2_ref[...], preferred_element_type=jnp.float32) \
        + b2_ref[...]
    out_ref[...] = _ln(x1 + h2, gamma, beta)


def _stage3(x, h, ws, pmsg, pden, bias_gat, gamma, beta, W1, b1, W2, b2):
    B = 2000
    return pl.pallas_call(
        _stage3_body,
        grid=(N // B,),
        in_specs=[
            pl.BlockSpec((B, D), lambda i: (i, 0)),
            pl.BlockSpec((B, D), lambda i: (i, 0)),
            pl.BlockSpec((B, 16), lambda i: (i, 0)),
            pl.BlockSpec((1, B, D), lambda i: (0, i, 0)),
            pl.BlockSpec((1, B, D), lambda i: (1, i, 0)),
            pl.BlockSpec((1, B, 16), lambda i: (0, i, 0)),
            pl.BlockSpec((1, B, 16), lambda i: (1, i, 0)),
            pl.BlockSpec((D,), lambda i: (0,)),
            pl.BlockSpec((D,), lambda i: (0,)),
            pl.BlockSpec((D,), lambda i: (0,)),
            pl.BlockSpec((D, 2 * D), lambda i: (0, 0)),
            pl.BlockSpec((2 * D,), lambda i: (0,)),
            pl.BlockSpec((2 * D, D), lambda i: (0, 0)),
            pl.BlockSpec((D,), lambda i: (0,)),
        ],
        out_specs=pl.BlockSpec((B, D), lambda i: (i, 0)),
        out_shape=jax.ShapeDtypeStruct((N, D), jnp.float32),
    )(x, h, ws, pmsg, pmsg, pden, pden, bias_gat, gamma, beta,
      W1, b1, W2, b2)


def kernel(x, edge_index, W_gat, att_src, att_dst, bias_gat, gamma, beta,
           W1, b1, W2, b2):
    h, h_bf, aspad, adpad, wself = _stage1(x, W_gat, att_src, att_dst)
    src = edge_index[0].reshape(E // CHUNK, CHUNK)
    dst = edge_index[1].reshape(E // CHUNK, CHUNK)
    zmsg = jnp.zeros((N_PAD, D), jnp.float32)
    zden = jnp.zeros((N_PAD, 16), jnp.float32)
    pmsg, pden = _stage2(h_bf, aspad, adpad, src, dst, zmsg, zden)
    return _stage3(x, h, wself, pmsg, pden, bias_gat, gamma, beta,
                   W1, b1, W2, b2)


# speedup vs baseline: 1.1211x; 1.0001x over previous
import functools
import jax
import jax.numpy as jnp
from jax import lax
from jax.experimental import pallas as pl
from jax.experimental.pallas import tpu as pltpu
from jax.experimental.pallas import tpu_sc as plsc

N = 10000
E = 320000
D = 128
H = 4
C = D // H

ACCW = 144
N_PAD = 10112
NEG = -1e30

NC = 2
NS = 16
NW = NC * NS
EDGES_PER_TILE = E // NW
CHUNK = 80
NCHUNK = EDGES_PER_TILE // CHUNK
ROWS_PER_TILE = N_PAD // NS
ZROWS = 32


def _ln(x, g, b):
    m = jnp.mean(x, axis=-1, keepdims=True)
    v = jnp.var(x, axis=-1, keepdims=True)
    return (x - m) * lax.rsqrt(v + 1e-5) * g + b


def _stage1_body(x_ref, wg_ref, asrc_ref, adst_ref, h_ref, hbf_ref, as_ref,
                 ad_ref, ws_ref):
    x = x_ref[...]
    h = jnp.dot(x, wg_ref[...], preferred_element_type=jnp.float32)
    h_ref[...] = h
    jj = lax.broadcasted_iota(jnp.int32, (D, D), 0)
    ss = lax.broadcasted_iota(jnp.int32, (D, D), 1)
    r = ss % C
    srcc = (ss // C) * C + jnp.where(r % 2 == 0, r // 2, C // 2 + r // 2)
    pmat = (jj == srcc).astype(jnp.float32)
    hbf_ref[...] = jnp.dot(h, pmat,
                           preferred_element_type=jnp.float32).astype(
                               jnp.bfloat16)
    rows = lax.broadcasted_iota(jnp.int32, (D, 16), 0) // C
    cols = lax.broadcasted_iota(jnp.int32, (D, 16), 1)
    msel = (rows == cols).astype(jnp.float32)
    a_s = jnp.dot(h * asrc_ref[...], msel, preferred_element_type=jnp.float32)
    a_d = jnp.dot(h * adst_ref[...], msel, preferred_element_type=jnp.float32)
    head = lax.broadcasted_iota(jnp.int32, a_s.shape, 1) < H
    a_s = jnp.where(head, a_s, NEG)
    a_d = jnp.where(head, a_d, NEG)
    as_ref[...] = a_s
    ad_ref[...] = a_d
    t = a_s + a_d
    ws_ref[...] = jnp.exp(jnp.maximum(t, 0.2 * t))


def _stage1(x, W_gat, att_src, att_dst):
    B = 2000
    return pl.pallas_call(
        _stage1_body,
        grid=(N // B,),
        in_specs=[
            pl.BlockSpec((B, D), lambda i: (i, 0)),
            pl.BlockSpec((D, D), lambda i: (0, 0)),
            pl.BlockSpec((D,), lambda i: (0,)),
            pl.BlockSpec((D,), lambda i: (0,)),
        ],
        out_specs=[
            pl.BlockSpec((B, D), lambda i: (i, 0)),
            pl.BlockSpec((B, D), lambda i: (i, 0)),
            pl.BlockSpec((B, 16), lambda i: (i, 0)),
            pl.BlockSpec((B, 16), lambda i: (i, 0)),
            pl.BlockSpec((B, 16), lambda i: (i, 0)),
        ],
        out_shape=[
            jax.ShapeDtypeStruct((N, D), jnp.float32),
            jax.ShapeDtypeStruct((N, D), jnp.bfloat16),
            jax.ShapeDtypeStruct((N, 16), jnp.float32),
            jax.ShapeDtypeStruct((N, 16), jnp.float32),
            jax.ShapeDtypeStruct((N, 16), jnp.float32),
        ],
    )(x, W_gat, att_src.reshape(D), att_dst.reshape(D))


SUP = 5
NSUP = NCHUNK // SUP


def _bcast_lane(vec, lane):
    idx = jnp.full((16, 1), lane, jnp.int32)
    return lax.gather(
        vec, idx,
        lax.GatherDimensionNumbers(offset_dims=(), collapsed_slice_dims=(0,),
                                   start_index_map=(0,)),
        (1,), mode=lax.GatherScatterMode.PROMISE_IN_BOUNDS)


def _make_stage2():
    mesh = plsc.VectorSubcoreMesh(core_axis_name="c", subcore_axis_name="s")

    bufset = [
        pltpu.VMEM((CHUNK, 16), jnp.float32),
        pltpu.VMEM((CHUNK, 16), jnp.float32),
        pltpu.VMEM((CHUNK, 16), jnp.float32),
        pltpu.VMEM((CHUNK, D), jnp.bfloat16),
        pltpu.SemaphoreType.DMA,
        pltpu.SemaphoreType.DMA,
        pltpu.SemaphoreType.DMA,
        pltpu.SemaphoreType.DMA,
    ]
    idxset = [
        pltpu.VMEM((SUP, CHUNK), jnp.int32),
        pltpu.VMEM((SUP, CHUNK), jnp.int32),
        pltpu.SemaphoreType.DMA,
        pltpu.SemaphoreType.DMA,
    ]

    @functools.partial(
        pl.kernel,
        out_type=[
            jax.ShapeDtypeStruct((NC, N_PAD, D), jnp.float32),
            jax.ShapeDtypeStruct((NC, N_PAD, 16), jnp.float32),
        ],
        mesh=mesh,
        compiler_params=pltpu.CompilerParams(use_tc_tiling_on_sc=False,
                                             needs_layout_passes=False),
        scratch_types=bufset + bufset + idxset + idxset + [
            pltpu.VMEM((CHUNK, D), jnp.float32),
            pltpu.VMEM((CHUNK, D), jnp.float32),
            pltpu.SemaphoreType.DMA,
            pltpu.SemaphoreType.DMA,
            pltpu.VMEM_SHARED((N_PAD, D), jnp.float32),
            pltpu.VMEM_SHARED((N_PAD, 16), jnp.float32),
        ],
    )
    def stage2(h_hbm, aspad_hbm, adpad_hbm, src2d_hbm, dst2d_hbm,
               zmsg_hbm, zden_hbm,
               omsg_hbm, oden_hbm,
               asrcA, adstA, wbufA, rinA, semA1, semA2, semA3, semA4,
               asrcB, adstB, wbufB, rinB, semB1, semB2, semB3, semB4,
               sidx0, didx0, is0s, is0d,
               sidx1, didx1, is1s, is1d,
               routA, routB, smA, smB, msg_sh, den_sh):
        c = lax.axis_index("c")
        s = lax.axis_index("s")
        wid = c * NS + s
        bufs_a = (asrcA, adstA, wbufA, rinA, semA1, semA2, semA3, semA4,
                  routA, smA)
        bufs_b = (asrcB, adstB, wbufB, rinB, semB1, semB2, semB3, semB4,
                  routB, smB)
        idx0 = (sidx0, didx0, is0s, is0d)
        idx1 = (sidx1, didx1, is1s, is1d)

        off = s * ROWS_PER_TILE
        pltpu.sync_copy(zmsg_hbm.at[pl.ds(off, ROWS_PER_TILE)],
                        msg_sh.at[pl.ds(off, ROWS_PER_TILE)])
        pltpu.sync_copy(zden_hbm.at[pl.ds(off, ROWS_PER_TILE)],
                        den_sh.at[pl.ds(off, ROWS_PER_TILE)])
        plsc.subcore_barrier()

        def load_super(sup, ib):
            sb, db, ss, ds_ = ib
            r0 = wid * NCHUNK + sup * SUP
            pltpu.async_copy(src2d_hbm.at[pl.ds(r0, SUP)], sb, ss)
            pltpu.async_copy(dst2d_hbm.at[pl.ds(r0, SUP)], db, ds_)

        def wait_super(sup, ib):
            sb, db, ss, ds_ = ib
            r0 = wid * NCHUNK + sup * SUP
            pltpu.make_async_copy(src2d_hbm.at[pl.ds(r0, SUP)], sb, ss).wait()
            pltpu.make_async_copy(dst2d_hbm.at[pl.ds(r0, SUP)], db, ds_).wait()

        def issue(srow, drow, bufs):
            asrc_v, adst_v, _, rin_v, s1, s2, s3, s4 = bufs[:8]
            pltpu.async_copy(h_hbm.at[srow], rin_v, s1)
            pltpu.async_copy(aspad_hbm.at[srow], asrc_v, s2)
            pltpu.async_copy(adpad_hbm.at[drow], adst_v, s3)

        def process(srow, drow, bufs, dsem_wait):
            asrc_v, adst_v, wbuf_v, rin_v, s1, s2, s3, s4, rout_v, sm = bufs
            if dsem_wait is not None:
                pltpu.make_async_copy(wbuf_v, den_sh.at[dsem_wait], s4).wait()
                pltpu.make_async_copy(rout_v, msg_sh.at[dsem_wait], sm).wait()
            pltpu.make_async_copy(aspad_hbm.at[srow], asrc_v, s2).wait()
            pltpu.make_async_copy(adpad_hbm.at[drow], adst_v, s3).wait()
            pltpu.make_async_copy(h_hbm.at[srow], rin_v, s1).wait()

            @plsc.parallel_loop(0, CHUNK, unroll=8)
            def _edge(e):
                t = asrc_v[e, :] + adst_v[e, :]
                w = jnp.exp(jnp.maximum(t, 0.2 * t))
                wbuf_v[e, :] = w
                for hh in range(H):
                    w_spl = _bcast_lane(w, hh)
                    xa, xb = plsc.unpack(
                        rin_v[e, pl.ds(hh * C, C)],
                        format=plsc.PackFormat.INTERLEAVED)
                    rout_v[e, pl.ds(hh * C, 16)] = xa * w_spl
                    rout_v[e, pl.ds(hh * C + 16, 16)] = xb * w_spl

            pltpu.async_copy(wbuf_v, den_sh.at[drow], s4, add=True)
            pltpu.async_copy(rout_v, msg_sh.at[drow], sm, add=True)

        wait0 = load_super(0, idx0)
        wait_super(0, idx0)
        issue(sidx0.at[0], didx0.at[0], bufs_a)

        def body(j, carry, first=False):
            load_super(2 * j + 1, idx1)
            for k in range(10):
                half = k // 5
                r = k % 5
                ib = idx0 if half == 0 else idx1
                cur = bufs_a if k % 2 == 0 else bufs_b
                nxt = bufs_b if k % 2 == 0 else bufs_a
                if k == 4:
                    wait_super(2 * j + 1, idx1)
                if k == 5:
                    load_super(2 * j + 2, idx0)
                if k == 9:
                    wait_super(2 * j + 2, idx0)
                nk = k + 1
                nib = idx0 if (nk // 5) % 2 == 0 else idx1
                nr = nk % 5
                issue(nib[0].at[nr], nib[1].at[nr], nxt)
                pk = k - 2
                if pk >= 0:
                    prow = (idx0 if (pk // 5) == 0 else idx1)[1].at[pk % 5]
                else:
                    prow = (idx1 if (pk + 10) // 5 == 1 else idx0)[1].at[
                        (pk + 10) % 5]
                pdrow = None if first and k < 2 else prow
                process(ib[0].at[r], ib[1].at[r], cur, pdrow)
            return carry

        body(0, 0, first=True)
        lax.fori_loop(1, (NSUP - 1) // 2, body, 0)

        for k in range(SUP):
            cur = bufs_a if k % 2 == 0 else bufs_b
            nxt = bufs_b if k % 2 == 0 else bufs_a
            if k < SUP - 1:
                issue(sidx0.at[k + 1], didx0.at[k + 1], nxt)
            pk = k - 2
            if pk >= 0:
                prow = didx0.at[pk]
            else:
                prow = didx1.at[pk + 10 - 5]
            process(sidx0.at[k], didx0.at[k], cur, prow)
        pltpu.make_async_copy(bufs_b[2], den_sh.at[didx0.at[SUP - 2]],
                              bufs_b[7]).wait()
        pltpu.make_async_copy(bufs_a[2], den_sh.at[didx0.at[SUP - 1]],
                              bufs_a[7]).wait()
        pltpu.make_async_copy(bufs_b[8], msg_sh.at[didx0.at[SUP - 2]],
                              bufs_b[9]).wait()
        pltpu.make_async_copy(bufs_a[8], msg_sh.at[didx0.at[SUP - 1]],
                              bufs_a[9]).wait()

        plsc.subcore_barrier()
        off = s * ROWS_PER_TILE
        pltpu.sync_copy(msg_sh.at[pl.ds(off, ROWS_PER_TILE)],
                        omsg_hbm.at[c, pl.ds(off, ROWS_PER_TILE)])
        pltpu.sync_copy(den_sh.at[pl.ds(off, ROWS_PER_TILE)],
                        oden_hbm.at[c, pl.ds(off, ROWS_PER_TILE)])

    return stage2


_stage2 = _make_stage2()


def _stage3_body(x_ref, h_ref, ws_ref, pm0_ref, pm1_ref, pd0_ref, pd1_ref,
                 bias_ref, gamma_ref, beta_ref, W1_ref, b1_ref, W2_ref,
                 b2_ref, out_ref):
    x = x_ref[...]
    h = h_ref[...]
    ws = ws_ref[...]
    rows = lax.broadcasted_iota(jnp.int32, (16, D), 0)
    cols = lax.broadcasted_iota(jnp.int32, (16, D), 1) // C
    m16 = (rows == cols).astype(jnp.float32)
    den16 = pd0_ref[0] + pd1_ref[0] + ws
    den = jnp.dot(den16, m16, preferred_element_type=jnp.float32)
    ws128 = jnp.dot(ws, m16, preferred_element_type=jnp.float32)
    acc = pm0_ref[0] + pm1_ref[0] + h * ws128
    gat = acc / (den + 1e-16) + bias_ref[...]
    gamma = gamma_ref[...]
    beta = beta_ref[...]
    x1 = _ln(x + gat, gamma, beta)
    hmid = jnp.maximum(
        jnp.dot(x1, W1_ref[...], preferred_element_type=jnp.float32)
        + b1_ref[...], 0.0)
    h2 = jnp.dot(hmid, W2_ref[...], preferred_element_type=jnp.float32) \
        + b2_ref[...]
    out_ref[...] = _ln(x1 + h2, gamma, beta)


def _stage3(x, h, ws, pmsg, pden, bias_gat, gamma, beta, W1, b1, W2, b2):
    B = 2000
    return pl.pallas_call(
        _stage3_body,
        grid=(N // B,),
        in_specs=[
            pl.BlockSpec((B, D), lambda i: (i, 0)),
            pl.BlockSpec((B, D), lambda i: (i, 0)),
            pl.BlockSpec((B, 16), lambda i: (i, 0)),
            pl.BlockSpec((1, B, D), lambda i: (0, i, 0)),
            pl.BlockSpec((1, B, D), lambda i: (1, i, 0)),
            pl.BlockSpec((1, B, 16), lambda i: (0, i, 0)),
            pl.BlockSpec((1, B, 16), lambda i: (1, i, 0)),
            pl.BlockSpec((D,), lambda i: (0,)),
            pl.BlockSpec((D,), lambda i: (0,)),
            pl.BlockSpec((D,), lambda i: (0,)),
            pl.BlockSpec((D, 2 * D), lambda i: (0, 0)),
            pl.BlockSpec((2 * D,), lambda i: (0,)),
            pl.BlockSpec((2 * D, D), lambda i: (0, 0)),
            pl.BlockSpec((D,), lambda i: (0,)),
        ],
        out_specs=pl.BlockSpec((B, D), lambda i: (i, 0)),
        out_shape=jax.ShapeDtypeStruct((N, D), jnp.float32),
    )(x, h, ws, pmsg, pmsg, pden, pden, bias_gat, gamma, beta,
      W1, b1, W2, b2)


def kernel(x, edge_index, W_gat, att_src, att_dst, bias_gat, gamma, beta,
           W1, b1, W2, b2):
    h, h_bf, aspad, adpad, wself = _stage1(x, W_gat, att_src, att_dst)
    src = edge_index[0].reshape(E // CHUNK, CHUNK)
    dst = edge_index[1].reshape(E // CHUNK, CHUNK)
    zmsg = jnp.zeros((N_PAD, D), jnp.float32)
    zden = jnp.zeros((N_PAD, 16), jnp.float32)
    pmsg, pden = _stage2(h_bf, aspad, adpad, src, dst, zmsg, zden)
    return _stage3(x, h, wself, pmsg, pden, bias_gat, gamma, beta,
                   W1, b1, W2, b2)
